# baseline (device time: 86337 ns/iter reference)
import jax
import jax.numpy as jnp
from jax import lax
from jax.experimental import pallas as pl
from jax.experimental.pallas import tpu as pltpu

N_DEV = 4
SQ = 256
D_MODEL = 1024
H = 8
DH = 128
KWIN = 512
KMAX = 1152
PAD = 128
SCALE = 0.08838834764831843
NEG = -1e9


def kernel(x, Wq, K_ext, V_ext, Wo):
    i = lax.axis_index("i")
    K_h = lax.dynamic_slice_in_dim(K_ext[0, :KMAX], i * H, H, axis=1)
    V_h = lax.dynamic_slice_in_dim(V_ext[0, :KMAX], i * H, H, axis=1)
    K_pad = jnp.pad(K_h, ((PAD, 0), (0, 0), (0, 0)))
    V_pad = jnp.pad(V_h, ((PAD, 0), (0, 0), (0, 0)))

    def body(x_ref, wq_ref, k_ref, v_ref, wo_ref, out_ref,
             x_rel, partial, rs_buf, ag_send, ag_recv, rs_send, rs_recv):
        my = lax.axis_index("i")

        barrier = pltpu.get_barrier_semaphore()
        for d in range(1, N_DEV):
            pl.semaphore_signal(
                barrier, inc=1,
                device_id=((my + d) % N_DEV,),
                device_id_type=pl.DeviceIdType.MESH,
            )
        pl.semaphore_wait(barrier, N_DEV - 1)

        x_rel[0] = x_ref[0]

        ag = []
        for d in range(1, N_DEV):
            desc = pltpu.make_async_remote_copy(
                src_ref=x_rel.at[0],
                dst_ref=x_rel.at[d],
                send_sem=ag_send.at[d - 1],
                recv_sem=ag_recv.at[d - 1],
                device_id=((my + d) % N_DEV,),
                device_id_type=pl.DeviceIdType.MESH,
            )
            desc.start()
            ag.append(desc)

        row = lax.broadcasted_iota(jnp.int32, (SQ, KWIN), 0)
        col = lax.broadcasted_iota(jnp.int32, (SQ, KWIN), 1)
        window = (col >= row) & (col <= row + 2 * PAD)

        def compute_chunk(r):
            j = lax.rem(my + N_DEV - r, N_DEV)
            off = j * SQ
            q = lax.dot_general(
                x_rel[r], wq_ref[...], (((1,), (0,)), ((), ())),
                preferred_element_type=jnp.float32,
            )
            mask = window & ((col + off) >= PAD)
            ctx = []
            for h in range(H):
                qh = q[:, h * DH:(h + 1) * DH]
                kh = k_ref[pl.ds(off, KWIN), h, :]
                vh = v_ref[pl.ds(off, KWIN), h, :]
                s = lax.dot_general(
                    qh, kh, (((1,), (1,)), ((), ())),
                    preferred_element_type=jnp.float32,
                ) * SCALE
                s = jnp.where(mask, s, NEG)
                m = jnp.max(s, axis=1, keepdims=True)
                e = jnp.exp(s - m)
                w = e / jnp.sum(e, axis=1, keepdims=True)
                ctx.append(lax.dot_general(
                    w, vh, (((1,), (0,)), ((), ())),
                    preferred_element_type=jnp.float32,
                ))
            ctx = jnp.concatenate(ctx, axis=1)
            return lax.dot_general(
                ctx, wo_ref[...], (((1,), (0,)), ((), ())),
                preferred_element_type=jnp.float32,
            )

        partial[0] = compute_chunk(0)

        rs = []
        for r in range(1, N_DEV):
            ag[r - 1].wait_recv()
            partial[r] = compute_chunk(r)
            desc = pltpu.make_async_remote_copy(
                src_ref=partial.at[r],
                dst_ref=rs_buf.at[r - 1],
                send_sem=rs_send.at[r - 1],
                recv_sem=rs_recv.at[r - 1],
                device_id=((my + N_DEV - r) % N_DEV,),
                device_id_type=pl.DeviceIdType.MESH,
            )
            desc.start()
            rs.append(desc)

        for desc in rs:
            desc.wait_recv()
        out_ref[0] = partial[0] + rs_buf[0] + rs_buf[1] + rs_buf[2]

        for desc in ag:
            desc.wait_send()
        for desc in rs:
            desc.wait_send()

    return pl.pallas_call(
        body,
        out_shape=jax.ShapeDtypeStruct((1, SQ, D_MODEL), jnp.float32),
        in_specs=[pl.BlockSpec(memory_space=pltpu.VMEM)] * 5,
        out_specs=pl.BlockSpec(memory_space=pltpu.VMEM),
        scratch_shapes=[
            pltpu.VMEM((N_DEV, SQ, D_MODEL), jnp.float32),
            pltpu.VMEM((N_DEV, SQ, D_MODEL), jnp.float32),
            pltpu.VMEM((N_DEV - 1, SQ, D_MODEL), jnp.float32),
            pltpu.SemaphoreType.DMA((N_DEV - 1,)),
            pltpu.SemaphoreType.DMA((N_DEV - 1,)),
            pltpu.SemaphoreType.DMA((N_DEV - 1,)),
            pltpu.SemaphoreType.DMA((N_DEV - 1,)),
        ],
        compiler_params=pltpu.CompilerParams(collective_id=0),
    )(x, Wq, K_pad, V_pad, Wo)


# device time: 65304 ns/iter; 1.3221x vs baseline; 1.3221x over previous
import jax
import jax.numpy as jnp
from jax import lax
from jax.experimental import pallas as pl
from jax.experimental.pallas import tpu as pltpu

N_DEV = 4
SQ = 256
D_MODEL = 1024
H = 8
DH = 128
KWIN = 512
KMAX = 1152
PAD = 128
SCALE = 0.08838834764831843
NEG = -1e9


def kernel(x, Wq, K_ext, V_ext, Wo):
    def body(x_ref, wq_ref, k_hbm, v_hbm, wo_ref, out_ref,
             k_vmem, v_vmem, x_rel, partial, rs_buf,
             kv_sems, ag_send, ag_recv, rs_send, rs_recv):
        my = lax.axis_index("i")

        v_vmem[0:PAD] = jnp.zeros((PAD, H, DH), jnp.float32)

        kcopy = pltpu.make_async_copy(
            k_hbm.at[0, pl.ds(0, KMAX), pl.ds(my * H, H), :],
            k_vmem.at[pl.ds(PAD, KMAX)],
            kv_sems.at[0],
        )
        vcopy = pltpu.make_async_copy(
            v_hbm.at[0, pl.ds(0, KMAX), pl.ds(my * H, H), :],
            v_vmem.at[pl.ds(PAD, KMAX)],
            kv_sems.at[1],
        )
        kcopy.start()
        vcopy.start()

        barrier = pltpu.get_barrier_semaphore()
        for d in range(1, N_DEV):
            pl.semaphore_signal(
                barrier, inc=1,
                device_id=((my + d) % N_DEV,),
                device_id_type=pl.DeviceIdType.MESH,
            )
        pl.semaphore_wait(barrier, N_DEV - 1)

        x_rel[0] = x_ref[0]

        ag = []
        for d in range(1, N_DEV):
            desc = pltpu.make_async_remote_copy(
                src_ref=x_rel.at[0],
                dst_ref=x_rel.at[d],
                send_sem=ag_send.at[d - 1],
                recv_sem=ag_recv.at[d - 1],
                device_id=((my + d) % N_DEV,),
                device_id_type=pl.DeviceIdType.MESH,
            )
            desc.start()
            ag.append(desc)

        row = lax.broadcasted_iota(jnp.int32, (SQ, KWIN), 0)
        col = lax.broadcasted_iota(jnp.int32, (SQ, KWIN), 1)
        window = (col >= row) & (col <= row + 2 * PAD)

        def project_q(r):
            return lax.dot_general(
                x_rel[r], wq_ref[...], (((1,), (0,)), ((), ())),
                preferred_element_type=jnp.float32,
            )

        def attn_out(q, r):
            j = lax.rem(my + N_DEV - r, N_DEV)
            off = j * SQ
            mask = window & ((col + off) >= PAD)
            ctx = []
            for h in range(H):
                qh = q[:, h * DH:(h + 1) * DH]
                kh = k_vmem[pl.ds(off, KWIN), h, :]
                vh = v_vmem[pl.ds(off, KWIN), h, :]
                s = lax.dot_general(
                    qh, kh, (((1,), (1,)), ((), ())),
                    preferred_element_type=jnp.float32,
                ) * SCALE
                s = jnp.where(mask, s, NEG)
                m = jnp.max(s, axis=1, keepdims=True)
                e = jnp.exp(s - m)
                w = e / jnp.sum(e, axis=1, keepdims=True)
                ctx.append(lax.dot_general(
                    w, vh, (((1,), (0,)), ((), ())),
                    preferred_element_type=jnp.float32,
                ))
            ctx = jnp.concatenate(ctx, axis=1)
            return lax.dot_general(
                ctx, wo_ref[...], (((1,), (0,)), ((), ())),
                preferred_element_type=jnp.float32,
            )

        q0 = project_q(0)
        kcopy.wait()
        vcopy.wait()
        partial[0] = attn_out(q0, 0)

        rs = []
        for r in range(1, N_DEV):
            ag[r - 1].wait_recv()
            partial[r] = attn_out(project_q(r), r)
            desc = pltpu.make_async_remote_copy(
                src_ref=partial.at[r],
                dst_ref=rs_buf.at[r - 1],
                send_sem=rs_send.at[r - 1],
                recv_sem=rs_recv.at[r - 1],
                device_id=((my + N_DEV - r) % N_DEV,),
                device_id_type=pl.DeviceIdType.MESH,
            )
            desc.start()
            rs.append(desc)

        for desc in rs:
            desc.wait_recv()
        out_ref[0] = partial[0] + rs_buf[0] + rs_buf[1] + rs_buf[2]

        for desc in ag:
            desc.wait_send()
        for desc in rs:
            desc.wait_send()

    return pl.pallas_call(
        body,
        out_shape=jax.ShapeDtypeStruct((1, SQ, D_MODEL), jnp.float32),
        in_specs=[
            pl.BlockSpec(memory_space=pltpu.VMEM),
            pl.BlockSpec(memory_space=pltpu.VMEM),
            pl.BlockSpec(memory_space=pltpu.HBM),
            pl.BlockSpec(memory_space=pltpu.HBM),
            pl.BlockSpec(memory_space=pltpu.VMEM),
        ],
        out_specs=pl.BlockSpec(memory_space=pltpu.VMEM),
        scratch_shapes=[
            pltpu.VMEM((KMAX + PAD, H, DH), jnp.float32),
            pltpu.VMEM((KMAX + PAD, H, DH), jnp.float32),
            pltpu.VMEM((N_DEV, SQ, D_MODEL), jnp.float32),
            pltpu.VMEM((N_DEV, SQ, D_MODEL), jnp.float32),
            pltpu.VMEM((N_DEV - 1, SQ, D_MODEL), jnp.float32),
            pltpu.SemaphoreType.DMA((2,)),
            pltpu.SemaphoreType.DMA((N_DEV - 1,)),
            pltpu.SemaphoreType.DMA((N_DEV - 1,)),
            pltpu.SemaphoreType.DMA((N_DEV - 1,)),
            pltpu.SemaphoreType.DMA((N_DEV - 1,)),
        ],
        compiler_params=pltpu.CompilerParams(collective_id=0),
    )(x, Wq, K_ext, V_ext, Wo)


# device time: 41930 ns/iter; 2.0591x vs baseline; 1.5575x over previous
import jax
import jax.numpy as jnp
from jax import lax
from jax.experimental import pallas as pl
from jax.experimental.pallas import tpu as pltpu

N_DEV = 4
SQ = 256
D_MODEL = 1024
H = 8
DH = 128
KWIN = 512
KMAX = 1152
PAD = 128
SCALE = 0.08838834764831843
NEG = -1e9


def kernel(x, Wq, K_ext, V_ext, Wo):
    def body(x_hbm, wq_hbm, k_hbm, v_hbm, wo_hbm, out_ref,
             x_vmem, wq_vmem, k_vmem, v_vmem, wo_vmem,
             x_rel, partial, rs_buf,
             in_sems, ag_send, ag_recv, rs_send, rs_recv):
        my = lax.axis_index("i")

        v_vmem[0:PAD] = jnp.zeros((PAD, H, DH), jnp.float32)

        xcopy = pltpu.make_async_copy(x_hbm.at[0], x_vmem, in_sems.at[0])
        wqcopy = pltpu.make_async_copy(wq_hbm, wq_vmem, in_sems.at[1])
        wocopy = pltpu.make_async_copy(wo_hbm, wo_vmem, in_sems.at[2])
        kcopy = pltpu.make_async_copy(
            k_hbm.at[0, pl.ds(0, KMAX), pl.ds(my * H, H), :],
            k_vmem.at[pl.ds(PAD, KMAX)],
            in_sems.at[3],
        )
        vcopy = pltpu.make_async_copy(
            v_hbm.at[0, pl.ds(0, KMAX), pl.ds(my * H, H), :],
            v_vmem.at[pl.ds(PAD, KMAX)],
            in_sems.at[4],
        )
        xcopy.start()
        wqcopy.start()
        wocopy.start()
        kcopy.start()
        vcopy.start()

        barrier = pltpu.get_barrier_semaphore()
        for d in range(1, N_DEV):
            pl.semaphore_signal(
                barrier, inc=1,
                device_id=((my + d) % N_DEV,),
                device_id_type=pl.DeviceIdType.MESH,
            )
        pl.semaphore_wait(barrier, N_DEV - 1)

        xcopy.wait()
        x_rel[0] = x_vmem[...].astype(jnp.bfloat16)

        ag = []
        for d in range(1, N_DEV):
            desc = pltpu.make_async_remote_copy(
                src_ref=x_rel.at[0],
                dst_ref=x_rel.at[d],
                send_sem=ag_send.at[d - 1],
                recv_sem=ag_recv.at[d - 1],
                device_id=((my + d) % N_DEV,),
                device_id_type=pl.DeviceIdType.MESH,
            )
            desc.start()
            ag.append(desc)

        row = lax.broadcasted_iota(jnp.int32, (SQ, KWIN), 0)
        col = lax.broadcasted_iota(jnp.int32, (SQ, KWIN), 1)
        window = (col >= row) & (col <= row + 2 * PAD)

        def project_q(r):
            return lax.dot_general(
                x_rel[r].astype(jnp.float32), wq_vmem[...],
                (((1,), (0,)), ((), ())),
                preferred_element_type=jnp.float32,
            ) * SCALE

        def attn_out(q, r):
            j = lax.rem(my + N_DEV - r, N_DEV)
            off = j * SQ
            bias = jnp.where(window & ((col + off) >= PAD), 0.0, NEG)
            ctx = []
            for h in range(H):
                qh = q[:, h * DH:(h + 1) * DH]
                kh = k_vmem[pl.ds(off, KWIN), h, :]
                vh = v_vmem[pl.ds(off, KWIN), h, :]
                s = lax.dot_general(
                    qh, kh, (((1,), (1,)), ((), ())),
                    preferred_element_type=jnp.float32,
                )
                e = jnp.exp(s + bias)
                den = jnp.sum(e, axis=1, keepdims=True)
                ctx.append(lax.dot_general(
                    e, vh, (((1,), (0,)), ((), ())),
                    preferred_element_type=jnp.float32,
                ) / den)
            ctx = jnp.concatenate(ctx, axis=1)
            return lax.dot_general(
                ctx, wo_vmem[...], (((1,), (0,)), ((), ())),
                preferred_element_type=jnp.float32,
            )

        wqcopy.wait()
        q0 = project_q(0)
        kcopy.wait()
        vcopy.wait()
        wocopy.wait()
        partial[0] = attn_out(q0, 0).astype(jnp.bfloat16)

        rs = []
        for r in range(1, N_DEV):
            ag[r - 1].wait_recv()
            partial[r] = attn_out(project_q(r), r).astype(jnp.bfloat16)
            desc = pltpu.make_async_remote_copy(
                src_ref=partial.at[r],
                dst_ref=rs_buf.at[r - 1],
                send_sem=rs_send.at[r - 1],
                recv_sem=rs_recv.at[r - 1],
                device_id=((my + N_DEV - r) % N_DEV,),
                device_id_type=pl.DeviceIdType.MESH,
            )
            desc.start()
            rs.append(desc)

        acc = partial[0].astype(jnp.float32)
        for r, desc in enumerate(rs):
            desc.wait_recv()
            acc = acc + rs_buf[r].astype(jnp.float32)
        out_ref[0] = acc

        for desc in ag:
            desc.wait_send()
        for desc in rs:
            desc.wait_send()

    return pl.pallas_call(
        body,
        out_shape=jax.ShapeDtypeStruct((1, SQ, D_MODEL), jnp.float32),
        in_specs=[pl.BlockSpec(memory_space=pltpu.HBM)] * 5,
        out_specs=pl.BlockSpec(memory_space=pltpu.VMEM),
        scratch_shapes=[
            pltpu.VMEM((SQ, D_MODEL), jnp.float32),
            pltpu.VMEM((D_MODEL, D_MODEL), jnp.float32),
            pltpu.VMEM((KMAX + PAD, H, DH), jnp.float32),
            pltpu.VMEM((KMAX + PAD, H, DH), jnp.float32),
            pltpu.VMEM((D_MODEL, D_MODEL), jnp.float32),
            pltpu.VMEM((N_DEV, SQ, D_MODEL), jnp.bfloat16),
            pltpu.VMEM((N_DEV, SQ, D_MODEL), jnp.bfloat16),
            pltpu.VMEM((N_DEV - 1, SQ, D_MODEL), jnp.bfloat16),
            pltpu.SemaphoreType.DMA((5,)),
            pltpu.SemaphoreType.DMA((N_DEV - 1,)),
            pltpu.SemaphoreType.DMA((N_DEV - 1,)),
            pltpu.SemaphoreType.DMA((N_DEV - 1,)),
            pltpu.SemaphoreType.DMA((N_DEV - 1,)),
        ],
        compiler_params=pltpu.CompilerParams(collective_id=0),
    )(x, Wq, K_ext, V_ext, Wo)
